# Initial kernel scaffold; baseline (speedup 1.0000x reference)
import jax
import jax.numpy as jnp
from jax.experimental import pallas as pl


def kernel(indices, W0, W1, W2, P0, P1, P2):
    raise NotImplementedError("write your pallas kernel here")


# speedup vs baseline: 6.9707x; 6.9707x over previous
import functools

import jax
import jax.numpy as jnp
from jax import lax
from jax.experimental import pallas as pl
from jax.experimental.pallas import tpu as pltpu
from jax.experimental.pallas import tpu_sc as plsc

R = 16
N = 100000
B = 16384
NC = 2
NS = 16
NW = NC * NS
BPW = B // NW
CH = 128
NCHUNK = BPW // CH
L = 16


def _tt_kernel(ix0, ix1, ix2, t1w, t1p, sml,
               out, parts,
               ix0c, ix1c, ix2c,
               r1w, r1p, sm0, sm2,
               outc, pscr, sem):
    wid = lax.axis_index("s") * NC + lax.axis_index("c")
    lane = lax.broadcasted_iota(jnp.int32, (L,), 0)
    jsplat = [jnp.full((L,), j, jnp.int32) for j in range(R)]
    bfly = [lane ^ k for k in (8, 4, 2, 1)]

    def lanesum(v):
        for perm in bfly:
            v = v + v.at[perm].get(mode="promise_in_bounds")
        return v

    def chunk_body(c, sums):
        s0, s1, s2 = sums
        base = pl.multiple_of(wid * BPW + c * CH, CH)
        pltpu.sync_copy(ix0.at[pl.ds(base, CH)], ix0c)
        pltpu.sync_copy(ix1.at[pl.ds(base, CH)], ix1c)
        pltpu.sync_copy(ix2.at[pl.ds(base, CH)], ix2c)
        cps = [
            pltpu.async_copy(t1w.at[ix1c], r1w, sem),
            pltpu.async_copy(t1p.at[ix1c], r1p, sem),
            pltpu.async_copy(sml.at[ix0c], sm0, sem),
            pltpu.async_copy(sml.at[ix2c], sm2, sem),
        ]
        for cp in cps:
            cp.wait()

        def group_body(g, carry):
            gs0, gs1, gs2 = carry
            gbase = g * L
            outvec = jnp.zeros((L,), jnp.float32)
            for bl in range(L):
                b = gbase + bl
                a0v = sm0[b, pl.ds(0, L)] * sm0[b, pl.ds(L, L)]
                a2v = sm2[b, pl.ds(2 * L, L)] * sm2[b, pl.ds(3 * L, L)]
                acc = jnp.zeros((L,), jnp.float32)
                for j in range(R):
                    t = r1w[b, pl.ds(j * L, L)] * r1p[b, pl.ds(j * L, L)]
                    gs1 = gs1 + t
                    aj = a0v.at[jsplat[j]].get(mode="promise_in_bounds")
                    acc = acc + aj * t
                outval = lanesum(acc * a2v)
                outvec = jnp.where(lane == bl, outval, outvec)
                gs0 = gs0 + a0v
                gs2 = gs2 + a2v
            outc[pl.ds(gbase, L)] = outvec
            return (gs0, gs1, gs2)

        s0, s1, s2 = lax.fori_loop(0, CH // L, group_body, (s0, s1, s2))
        pltpu.sync_copy(outc, out.at[pl.ds(base, CH)])
        return (s0, s1, s2)

    z = jnp.zeros((L,), jnp.float32)
    s0, s1, s2 = lax.fori_loop(0, NCHUNK, chunk_body, (z, z, z))
    pscr[pl.ds(0, L)] = s0
    pscr[pl.ds(L, L)] = s1
    pscr[pl.ds(2 * L, L)] = s2
    pltpu.sync_copy(pscr, parts.at[wid])


@jax.jit
def kernel(indices, W0, W1, W2, P0, P1, P2):
    ix0 = indices[:, 0]
    ix1 = indices[:, 1]
    ix2 = indices[:, 2]
    t1w = jnp.transpose(W1, (1, 0, 2)).reshape(N, R * R)
    t1p = jnp.transpose(P1, (1, 0, 2)).reshape(N, R * R)
    sml = jnp.concatenate(
        [
            W0.reshape(N, R),
            P0.reshape(N, R),
            jnp.transpose(W2, (1, 0, 2)).reshape(N, R),
            jnp.transpose(P2, (1, 0, 2)).reshape(N, R),
            jnp.zeros((N, 64), jnp.float32),
        ],
        axis=1,
    )

    mesh = plsc.VectorSubcoreMesh(
        core_axis_name="c", subcore_axis_name="s", num_cores=NC, num_subcores=NS
    )
    run = functools.partial(
        pl.kernel,
        out_type=(
            jax.ShapeDtypeStruct((B,), jnp.float32),
            jax.ShapeDtypeStruct((NW, 3 * L), jnp.float32),
        ),
        mesh=mesh,
        compiler_params=pltpu.CompilerParams(use_tc_tiling_on_sc=True),
        scratch_types=[
            pltpu.VMEM((CH,), jnp.int32),
            pltpu.VMEM((CH,), jnp.int32),
            pltpu.VMEM((CH,), jnp.int32),
            pltpu.VMEM((CH, R * R), jnp.float32),
            pltpu.VMEM((CH, R * R), jnp.float32),
            pltpu.VMEM((CH, 128), jnp.float32),
            pltpu.VMEM((CH, 128), jnp.float32),
            pltpu.VMEM((CH,), jnp.float32),
            pltpu.VMEM((3 * L,), jnp.float32),
            pltpu.SemaphoreType.DMA,
        ],
    )(_tt_kernel)
    preds, parts = run(ix0, ix1, ix2, t1w, t1p, sml)

    s0 = jnp.sum(parts[:, 0:L])
    s1 = jnp.sum(parts[:, L:2 * L])
    s2 = jnp.sum(parts[:, 2 * L:3 * L])
    reg = 0.01 * (jnp.abs(s0) / (B * R) + jnp.abs(s1) / (B * R * R)
                  + jnp.abs(s2) / (B * R))
    return preds, reg.astype(jnp.float32)


# speedup vs baseline: 7.1273x; 1.0225x over previous
import functools

import jax
import jax.numpy as jnp
from jax import lax
from jax.experimental import pallas as pl
from jax.experimental.pallas import tpu as pltpu
from jax.experimental.pallas import tpu_sc as plsc

R = 16
N = 100000
B = 16384
NC = 2
NS = 16
NW = NC * NS
BPW = B // NW
CH = 64
NCHUNK = BPW // CH
L = 16


def _tt_kernel(ix0, ix1, ix2, t1w, t1p, sml,
               out, parts,
               ix0c, ix1c, ix2c,
               r1w, r1p, sm0, sm2,
               outc, pscr, sem0, sem1):
    wid = lax.axis_index("s") * NC + lax.axis_index("c")
    lane = lax.broadcasted_iota(jnp.int32, (L,), 0)
    jsplat = [jnp.full((L,), j, jnp.int32) for j in range(R)]
    bfly = [lane ^ k for k in (8, 4, 2, 1)]
    sems = (sem0, sem1)

    def lanesum(v):
        for perm in bfly:
            v = v + v.at[perm].get(mode="promise_in_bounds")
        return v

    def stage_and_fire(c, buf):
        base = pl.multiple_of(wid * BPW + c * CH, CH)
        pltpu.sync_copy(ix0.at[pl.ds(base, CH)], ix0c.at[buf])
        pltpu.sync_copy(ix1.at[pl.ds(base, CH)], ix1c.at[buf])
        pltpu.sync_copy(ix2.at[pl.ds(base, CH)], ix2c.at[buf])
        pltpu.async_copy(t1w.at[ix1c.at[buf]], r1w.at[buf], sems[buf])
        pltpu.async_copy(t1p.at[ix1c.at[buf]], r1p.at[buf], sems[buf])
        pltpu.async_copy(sml.at[ix0c.at[buf]], sm0.at[buf], sems[buf])
        pltpu.async_copy(sml.at[ix2c.at[buf]], sm2.at[buf], sems[buf])

    def drain(buf):
        pltpu.make_async_copy(t1w.at[ix1c.at[buf]], r1w.at[buf], sems[buf]).wait()
        pltpu.make_async_copy(t1p.at[ix1c.at[buf]], r1p.at[buf], sems[buf]).wait()
        pltpu.make_async_copy(sml.at[ix0c.at[buf]], sm0.at[buf], sems[buf]).wait()
        pltpu.make_async_copy(sml.at[ix2c.at[buf]], sm2.at[buf], sems[buf]).wait()

    def compute_chunk(c, buf, sums):
        s0, s1, s2 = sums
        base = pl.multiple_of(wid * BPW + c * CH, CH)

        def group_body(g, carry):
            gs0, gs1, gs2 = carry
            gbase = g * L
            outvec = jnp.zeros((L,), jnp.float32)
            for bl in range(L):
                b = gbase + bl
                a0v = sm0[buf, b, pl.ds(0, L)] * sm0[buf, b, pl.ds(L, L)]
                a2v = sm2[buf, b, pl.ds(2 * L, L)] * sm2[buf, b, pl.ds(3 * L, L)]
                acc = jnp.zeros((L,), jnp.float32)
                for j in range(R):
                    t = r1w[buf, b, pl.ds(j * L, L)] * r1p[buf, b, pl.ds(j * L, L)]
                    gs1 = gs1 + t
                    aj = a0v.at[jsplat[j]].get(mode="promise_in_bounds")
                    acc = acc + aj * t
                outval = lanesum(acc * a2v)
                outvec = jnp.where(lane == bl, outval, outvec)
                gs0 = gs0 + a0v
                gs2 = gs2 + a2v
            outc[pl.ds(gbase, L)] = outvec
            return (gs0, gs1, gs2)

        s0, s1, s2 = lax.fori_loop(0, CH // L, group_body, (s0, s1, s2))
        pltpu.sync_copy(outc, out.at[pl.ds(base, CH)])
        return (s0, s1, s2)

    stage_and_fire(0, 0)

    def super_body(h, sums):
        for sub in range(2):
            c = h * 2 + sub
            nxt = c + 1
            @pl.when(nxt < NCHUNK)
            def _prefetch():
                stage_and_fire(nxt, (sub + 1) % 2)
            drain(sub)
            sums = compute_chunk(c, sub, sums)
        return sums

    z = jnp.zeros((L,), jnp.float32)
    s0, s1, s2 = lax.fori_loop(0, NCHUNK // 2, super_body, (z, z, z))
    pscr[pl.ds(0, L)] = s0
    pscr[pl.ds(L, L)] = s1
    pscr[pl.ds(2 * L, L)] = s2
    pltpu.sync_copy(pscr, parts.at[wid])


@jax.jit
def kernel(indices, W0, W1, W2, P0, P1, P2):
    ix0 = indices[:, 0]
    ix1 = indices[:, 1]
    ix2 = indices[:, 2]
    t1w = jnp.transpose(W1, (1, 0, 2)).reshape(N, R * R)
    t1p = jnp.transpose(P1, (1, 0, 2)).reshape(N, R * R)
    sml = jnp.concatenate(
        [
            W0.reshape(N, R),
            P0.reshape(N, R),
            jnp.transpose(W2, (1, 0, 2)).reshape(N, R),
            jnp.transpose(P2, (1, 0, 2)).reshape(N, R),
            jnp.zeros((N, 64), jnp.float32),
        ],
        axis=1,
    )

    mesh = plsc.VectorSubcoreMesh(
        core_axis_name="c", subcore_axis_name="s", num_cores=NC, num_subcores=NS
    )
    run = functools.partial(
        pl.kernel,
        out_type=(
            jax.ShapeDtypeStruct((B,), jnp.float32),
            jax.ShapeDtypeStruct((NW, 3 * L), jnp.float32),
        ),
        mesh=mesh,
        compiler_params=pltpu.CompilerParams(use_tc_tiling_on_sc=True),
        scratch_types=[
            pltpu.VMEM((2, CH), jnp.int32),
            pltpu.VMEM((2, CH), jnp.int32),
            pltpu.VMEM((2, CH), jnp.int32),
            pltpu.VMEM((2, CH, R * R), jnp.float32),
            pltpu.VMEM((2, CH, R * R), jnp.float32),
            pltpu.VMEM((2, CH, 128), jnp.float32),
            pltpu.VMEM((2, CH, 128), jnp.float32),
            pltpu.VMEM((CH,), jnp.float32),
            pltpu.VMEM((3 * L,), jnp.float32),
            pltpu.SemaphoreType.DMA,
            pltpu.SemaphoreType.DMA,
        ],
    )(_tt_kernel)
    preds, parts = run(ix0, ix1, ix2, t1w, t1p, sml)

    s0 = jnp.sum(parts[:, 0:L])
    s1 = jnp.sum(parts[:, L:2 * L])
    s2 = jnp.sum(parts[:, 2 * L:3 * L])
    reg = 0.01 * (jnp.abs(s0) / (B * R) + jnp.abs(s1) / (B * R * R)
                  + jnp.abs(s2) / (B * R))
    return preds, reg.astype(jnp.float32)
